# baseline (speedup 1.0000x reference)
import functools

import jax
import jax.numpy as jnp
from jax import lax
from jax.experimental import pallas as pl
from jax.experimental.pallas import tpu as pltpu
from jax.experimental.pallas import tpu_sc as plsc

B = 16384
EMB = 64
RW = 2 * EMB
NC = 2
NS = 16
NW = NC * NS
BPW = B // NW
CH = 128
NCHUNK = BPW // CH
NTAB = 3

MLP_BLK = 2048
TP_C = 16384


def _tp_body(x, y):
    xt = x[...].T
    x4 = xt.reshape(TP_C // 128, 2, 128, EMB)
    y[...] = jnp.concatenate([x4[:, 0], x4[:, 1]], axis=-1).reshape(TP_C, RW)


def _transpose_pack(tT):
    v = tT.shape[1]
    rows = ((v + 255) // 256) * 128
    grid = (pl.cdiv(rows, TP_C),)
    return pl.pallas_call(
        _tp_body,
        grid=grid,
        in_specs=[pl.BlockSpec((EMB, 2 * TP_C), lambda i: (0, i))],
        out_specs=pl.BlockSpec((TP_C, RW), lambda i: (i, 0)),
        out_shape=jax.ShapeDtypeStruct((rows, RW), jnp.float32),
    )(tT)


def _sc_gather(u_idx, i_idx, c_idx, u_tab, i_tab, c_tab):
    mesh = plsc.VectorSubcoreMesh(core_axis_name="c", subcore_axis_name="s")

    @functools.partial(
        pl.kernel,
        mesh=mesh,
        out_type=[jax.ShapeDtypeStruct((B, RW), jnp.float32)] * NTAB,
        scratch_types=[
            pltpu.VMEM((BPW,), jnp.int32),
            pltpu.VMEM((BPW,), jnp.int32),
            pltpu.VMEM((BPW,), jnp.int32),
            pltpu.VMEM((NCHUNK * CH, RW), jnp.float32),
        ] + [pltpu.SemaphoreType.DMA] * 8,
    )
    def k(uix, iix, cix, utab, itab, ctab, out_u, out_i, out_c,
          iv_u, iv_i, iv_c, rows, g0, g1, g2, g3, o0, o1, o2, o3):
        wid = lax.axis_index("s") * NC + lax.axis_index("c")
        base = wid * BPW
        ivs = (iv_u, iv_i, iv_c)
        tabs = (utab, itab, ctab)
        outs = (out_u, out_i, out_c)
        gsem = (g0, g1, g2, g3)
        osem = (o0, o1, o2, o3)
        pltpu.sync_copy(uix.at[pl.ds(base, BPW)], iv_u)
        pltpu.sync_copy(iix.at[pl.ds(base, BPW)], iv_i)
        pltpu.sync_copy(cix.at[pl.ds(base, BPW)], iv_c)
        nk = NTAB * NCHUNK
        gh = [None] * 4
        oh = [None] * 4
        for step in range(nk + 1):
            if step < nk:
                t, j, s = step // NCHUNK, step % NCHUNK, step % 4
                if oh[s] is not None:
                    oh[s].wait()
                gh[s] = pltpu.async_copy(
                    tabs[t].at[ivs[t].at[pl.ds(j * CH, CH)]],
                    rows.at[pl.ds(s * CH, CH)], gsem[s])
            if step >= 1:
                p = step - 1
                t, j, s = p // NCHUNK, p % NCHUNK, p % 4
                gh[s].wait()
                oh[s] = pltpu.async_copy(
                    rows.at[pl.ds(s * CH, CH)],
                    outs[t].at[pl.ds(base + j * CH, CH)], osem[s])
        for s in range(4):
            if oh[s] is not None:
                oh[s].wait()

    return k(u_idx, i_idx, c_idx, u_tab, i_tab, c_tab)


def _mlp_body(xu, xi, xc, pu, pi_, pc, w1u, w1i, w1c, b1, w2, b2, w3, b3,
              wo, bo, out):
    f32 = jnp.float32
    bf16 = jnp.bfloat16

    def sel(x, p):
        return x[:, :EMB] * (1.0 - p[...]) + x[:, EMB:] * p[...]

    def dot(a, w):
        return jnp.dot(a.astype(bf16), w.astype(bf16),
                       preferred_element_type=f32)

    h = (dot(sel(xu[...], pu), w1u[...])
         + dot(sel(xi[...], pi_), w1i[...])
         + dot(sel(xc[...], pc), w1c[...])
         + b1[...])
    h = jnp.maximum(h, 0.0)
    h = jnp.maximum(dot(h, w2[...]) + b2[...], 0.0)
    h = jnp.maximum(dot(h, w3[...]) + b3[...], 0.0)
    out[...] = dot(h, wo[...]) + bo[...]


def _mlp(xu, xi, xc, pu, pi_, pc, W1, b1, W2, b2, W3, b3, Wout, bout):
    grid = (B // MLP_BLK,)
    x_spec = pl.BlockSpec((MLP_BLK, RW), lambda i: (i, 0))
    p_spec = pl.BlockSpec((MLP_BLK, 1), lambda i: (i, 0))
    full = lambda r, c: pl.BlockSpec((r, c), lambda i: (0, 0))
    out2d = pl.pallas_call(
        _mlp_body,
        grid=grid,
        in_specs=[
            x_spec, x_spec, x_spec,
            p_spec, p_spec, p_spec,
            full(EMB, 128), full(EMB, 128), full(EMB, 128), full(1, 128),
            full(128, 64), full(1, 64),
            full(64, 32), full(1, 32),
            full(32, 1), full(1, 1),
        ],
        out_specs=pl.BlockSpec((MLP_BLK, 1), lambda i: (i, 0)),
        out_shape=jax.ShapeDtypeStruct((B, 1), jnp.float32),
    )(xu, xi, xc, pu, pi_, pc,
      W1[0:EMB], W1[EMB:2 * EMB], W1[2 * EMB:3 * EMB], b1.reshape(1, -1),
      W2, b2.reshape(1, -1), W3, b3.reshape(1, -1),
      Wout, bout.reshape(1, -1))
    return out2d[:, 0]


def kernel(user_input, item_input, category_input, user_table, item_table,
           cat_table, W1, b1, W2, b2, W3, b3, Wout, bout):
    f32 = jnp.float32
    iu = user_input.astype(jnp.int32)
    ii = item_input.astype(jnp.int32)
    ic = category_input.astype(jnp.int32)
    def pack_idx(x):
        return ((x >> 8) << 7) | (x & 127)

    pu = ((iu >> 7) & 1).astype(f32).reshape(B, 1)
    pi_ = ((ii >> 7) & 1).astype(f32).reshape(B, 1)
    pc = ((ic >> 7) & 1).astype(f32).reshape(B, 1)
    xu, xi, xc = _sc_gather(
        pack_idx(iu), pack_idx(ii), pack_idx(ic),
        _transpose_pack(user_table.T),
        _transpose_pack(item_table.T),
        _transpose_pack(cat_table.T))
    return _mlp(xu, xi, xc, pu, pi_, pc, W1, b1, W2, b2, W3, b3, Wout, bout)


# speedup vs baseline: 1.0212x; 1.0212x over previous
import functools

import jax
import jax.numpy as jnp
from jax import lax
from jax.experimental import pallas as pl
from jax.experimental.pallas import tpu as pltpu
from jax.experimental.pallas import tpu_sc as plsc

B = 16384
EMB = 64
RW = 2 * EMB
NC = 2
NS = 16
NW = NC * NS
BPW = B // NW
CH = 128
NCHUNK = BPW // CH
NTAB = 3

MLP_BLK = 2048
TP_C = 16384


def _tp_body(x, y):
    xt = x[...].T
    x4 = xt.reshape(TP_C // 128, 2, 128, EMB)
    y[...] = jnp.concatenate([x4[:, 0], x4[:, 1]], axis=-1).reshape(TP_C, RW)


def _transpose_pack(tT):
    v = tT.shape[1]
    rows = ((v + 255) // 256) * 128
    grid = (pl.cdiv(rows, TP_C),)
    return pl.pallas_call(
        _tp_body,
        grid=grid,
        in_specs=[pl.BlockSpec((EMB, 2 * TP_C), lambda i: (0, i))],
        out_specs=pl.BlockSpec((TP_C, RW), lambda i: (i, 0)),
        out_shape=jax.ShapeDtypeStruct((rows, RW), jnp.float32),
    )(tT)


def _sc_gather1(idx, tab):
    mesh = plsc.VectorSubcoreMesh(core_axis_name="c", subcore_axis_name="s")

    @functools.partial(
        pl.kernel,
        mesh=mesh,
        out_type=jax.ShapeDtypeStruct((B, RW), jnp.float32),
        scratch_types=[
            pltpu.VMEM((BPW,), jnp.int32),
            pltpu.VMEM((NCHUNK * CH, RW), jnp.float32),
        ] + [pltpu.SemaphoreType.DMA] * 8,
    )
    def k(ix, tb, out, iv, rows, g0, g1, g2, g3, o0, o1, o2, o3):
        wid = lax.axis_index("s") * NC + lax.axis_index("c")
        base = wid * BPW
        gsem = (g0, g1, g2, g3)
        osem = (o0, o1, o2, o3)
        pltpu.sync_copy(ix.at[pl.ds(base, BPW)], iv)
        gh = [None] * NCHUNK
        oh = [None] * NCHUNK
        for step in range(NCHUNK + 1):
            if step < NCHUNK:
                gh[step] = pltpu.async_copy(
                    tb.at[iv.at[pl.ds(step * CH, CH)]],
                    rows.at[pl.ds(step * CH, CH)], gsem[step])
            if step >= 1:
                p = step - 1
                gh[p].wait()
                oh[p] = pltpu.async_copy(
                    rows.at[pl.ds(p * CH, CH)],
                    out.at[pl.ds(base + p * CH, CH)], osem[p])
        for s in range(NCHUNK):
            if oh[s] is not None:
                oh[s].wait()

    return k(idx, tab)


def _mlp_body(xu, xi, xc, pu, pi_, pc, w1u, w1i, w1c, b1, w2, b2, w3, b3,
              wo, bo, out):
    f32 = jnp.float32
    bf16 = jnp.bfloat16

    def sel(x, p):
        return x[:, :EMB] * (1.0 - p[...]) + x[:, EMB:] * p[...]

    def dot(a, w):
        return jnp.dot(a.astype(bf16), w.astype(bf16),
                       preferred_element_type=f32)

    h = (dot(sel(xu[...], pu), w1u[...])
         + dot(sel(xi[...], pi_), w1i[...])
         + dot(sel(xc[...], pc), w1c[...])
         + b1[...])
    h = jnp.maximum(h, 0.0)
    h = jnp.maximum(dot(h, w2[...]) + b2[...], 0.0)
    h = jnp.maximum(dot(h, w3[...]) + b3[...], 0.0)
    out[...] = dot(h, wo[...]) + bo[...]


def _mlp(xu, xi, xc, pu, pi_, pc, W1, b1, W2, b2, W3, b3, Wout, bout):
    grid = (B // MLP_BLK,)
    x_spec = pl.BlockSpec((MLP_BLK, RW), lambda i: (i, 0))
    p_spec = pl.BlockSpec((MLP_BLK, 1), lambda i: (i, 0))
    full = lambda r, c: pl.BlockSpec((r, c), lambda i: (0, 0))
    out2d = pl.pallas_call(
        _mlp_body,
        grid=grid,
        in_specs=[
            x_spec, x_spec, x_spec,
            p_spec, p_spec, p_spec,
            full(EMB, 128), full(EMB, 128), full(EMB, 128), full(1, 128),
            full(128, 64), full(1, 64),
            full(64, 32), full(1, 32),
            full(32, 1), full(1, 1),
        ],
        out_specs=pl.BlockSpec((MLP_BLK, 1), lambda i: (i, 0)),
        out_shape=jax.ShapeDtypeStruct((B, 1), jnp.float32),
    )(xu, xi, xc, pu, pi_, pc,
      W1[0:EMB], W1[EMB:2 * EMB], W1[2 * EMB:3 * EMB], b1.reshape(1, -1),
      W2, b2.reshape(1, -1), W3, b3.reshape(1, -1),
      Wout, bout.reshape(1, -1))
    return out2d[:, 0]


def kernel(user_input, item_input, category_input, user_table, item_table,
           cat_table, W1, b1, W2, b2, W3, b3, Wout, bout):
    f32 = jnp.float32
    iu = user_input.astype(jnp.int32)
    ii = item_input.astype(jnp.int32)
    ic = category_input.astype(jnp.int32)
    def pack_idx(x):
        return ((x >> 8) << 7) | (x & 127)

    pu = ((iu >> 7) & 1).astype(f32).reshape(B, 1)
    pi_ = ((ii >> 7) & 1).astype(f32).reshape(B, 1)
    pc = ((ic >> 7) & 1).astype(f32).reshape(B, 1)
    xu = _sc_gather1(pack_idx(iu), _transpose_pack(user_table.T))
    xi = _sc_gather1(pack_idx(ii), _transpose_pack(item_table.T))
    xc = _sc_gather1(pack_idx(ic), _transpose_pack(cat_table.T))
    return _mlp(xu, xi, xc, pu, pi_, pc, W1, b1, W2, b2, W3, b3, Wout, bout)
